# baseline (speedup 1.0000x reference)
import functools

import numpy as np
import jax
import jax.numpy as jnp
from jax import lax
from jax.experimental import pallas as pl
from jax.experimental.pallas import tpu as pltpu
from jax.experimental.pallas import tpu_sc as plsc

NG = 512
NN = 100
NT = NG * NN
NE = NT * 16
D = 128

NC, NS = 2, 16
NW = NC * NS
HALF = NT // NC
CH = HALF // 2
CHW = CH * D
TRASH = CHW
SH_W = CHW + 2048
PTW = CHW // NS
ZB = 10240
EB_A = 6400
ESHARD = NE // NW
NSHARD = NT // NW
NK = NE + NT
KSHARD = NK // NS
KB = 3200
KR = KB // 128
NKB = KSHARD // KB

_IU, _JU = np.triu_indices(NN, k=1)
NTRI = _IU.size
OW = NTRI + 1
NTRI_PAD = 4960
TRI_IDX = np.zeros(NTRI_PAD, dtype=np.int32)
TRI_IDX[:NTRI] = _IU * NN + _JU
GPT = NG // NW
ORW = GPT * OW


def _sc_keys(nids, edge_index):
    mesh = plsc.VectorSubcoreMesh(
        core_axis_name="c", subcore_axis_name="s", num_cores=NC, num_subcores=NS)

    @functools.partial(
        pl.kernel,
        out_type=jax.ShapeDtypeStruct((NK,), jnp.int32),
        mesh=mesh,
        compiler_params=pltpu.CompilerParams(needs_layout_passes=False),
        scratch_types=[
            pltpu.VMEM((NT,), jnp.int32),
            pltpu.VMEM((EB_A,), jnp.int32),
            pltpu.VMEM((EB_A,), jnp.int32),
            pltpu.VMEM((EB_A,), jnp.int32),
        ],
    )
    def keys_kernel(nids_hbm, edge_hbm, out_hbm, nids_v, sbuf, dbuf, kbuf):
        c = lax.axis_index("c")
        s = lax.axis_index("s")
        w = s * NC + c
        iota16 = lax.iota(jnp.int32, 16)

        pltpu.sync_copy(nids_hbm, nids_v)

        for b in range(ESHARD // EB_A):
            ebase = w * ESHARD + b * EB_A
            pltpu.sync_copy(edge_hbm.at[0, pl.ds(ebase, EB_A)], sbuf)
            pltpu.sync_copy(edge_hbm.at[1, pl.ds(ebase, EB_A)], dbuf)

            @plsc.parallel_loop(0, EB_A // 16, unroll=8)
            def _(v):
                sv = sbuf[pl.ds(v * 16, 16)]
                dv = dbuf[pl.ds(v * 16, 16)]
                nv = plsc.load_gather(nids_v, [sv])
                kbuf[pl.ds(v * 16, 16)] = dv * D + nv
            pltpu.sync_copy(kbuf, out_hbm.at[pl.ds(ebase, EB_A)])

        nbase = w * NSHARD

        @plsc.parallel_loop(0, NSHARD // 16, unroll=8)
        def _(v):
            iv = nbase + v * 16 + iota16
            nv = plsc.load_gather(nids_v, [iv])
            kbuf[pl.ds(v * 16, 16)] = iv * D + nv
        pltpu.sync_copy(kbuf.at[pl.ds(0, NSHARD)],
                        out_hbm.at[pl.ds(NE + nbase, NSHARD)])

    return keys_kernel(nids, edge_index)


def _sc_histogram(keys):
    mesh = plsc.VectorSubcoreMesh(
        core_axis_name="c", subcore_axis_name="s", num_cores=NC, num_subcores=NS)

    @functools.partial(
        pl.kernel,
        out_type=jax.ShapeDtypeStruct((NT * D,), jnp.float32),
        mesh=mesh,
        compiler_params=pltpu.CompilerParams(needs_layout_passes=False),
        scratch_types=[
            pltpu.VMEM((KB,), jnp.int32),
            pltpu.VMEM((KB,), jnp.int32),
            pltpu.VMEM((KB,), jnp.int32),
            pltpu.VMEM((KB,), jnp.float32),
            pltpu.VMEM((ZB,), jnp.float32),
            pltpu.VMEM_SHARED((SH_W,), jnp.float32),
            pltpu.SemaphoreType.DMA,
        ],
    )
    def hist_kernel(keys_hbm, out_hbm, kbuf, idx_a, idx_b, ones1d, zeros_v,
                    shared, sem):
        c = lax.axis_index("c")
        s = lax.axis_index("s")

        def ones_body(v, _):
            ones1d[pl.ds(v * 16, 16)] = jnp.ones((16,), jnp.float32)
            return 0
        lax.fori_loop(0, KB // 16, ones_body, 0)

        def zbuf_body(i, _):
            zeros_v[pl.ds(i * 16, 16)] = jnp.zeros((16,), jnp.float32)
            return 0
        lax.fori_loop(0, ZB // 16, zbuf_body, 0)

        for chunk in range(2):
            lo = (c * HALF + chunk * CH) * D
            for z in range(PTW // ZB):
                pltpu.sync_copy(zeros_v,
                                shared.at[pl.ds(s * PTW + z * ZB, ZB)])
            plsc.subcore_barrier()

            descs = [None, None]
            for b in range(NKB):
                kbase = s * KSHARD + b * KB
                pltpu.sync_copy(keys_hbm.at[pl.ds(kbase, KB)], kbuf)
                ib = idx_a if b % 2 == 0 else idx_b
                if descs[b % 2] is not None:
                    descs[b % 2].wait()

                @plsc.parallel_loop(0, KB // 16, unroll=8)
                def _(v):
                    kv = kbuf[pl.ds(v * 16, 16)]
                    rel = kv - lo
                    ok = (rel >= 0) & (rel < CHW)
                    ib[pl.ds(v * 16, 16)] = jnp.where(ok, rel, -1)
                descs[b % 2] = pltpu.async_copy(
                    ones1d,
                    shared.at[plsc.Indices(ib, ignored_value=-1)],
                    sem, add=True)
            for dsc in descs:
                if dsc is not None:
                    dsc.wait()

            plsc.subcore_barrier()
            pltpu.sync_copy(shared.at[pl.ds(s * PTW, PTW)],
                            out_hbm.at[pl.ds(lo + s * PTW, PTW)])

    return hist_kernel(keys)


def _sc_extract(dots_flat, exit_flat, tri):
    mesh = plsc.VectorSubcoreMesh(
        core_axis_name="c", subcore_axis_name="s", num_cores=NC, num_subcores=NS)

    @functools.partial(
        pl.kernel,
        out_type=jax.ShapeDtypeStruct((NG * OW,), jnp.float32),
        mesh=mesh,
        compiler_params=pltpu.CompilerParams(needs_layout_passes=False),
        scratch_types=[
            pltpu.VMEM((NN * NN,), jnp.float32),
            pltpu.VMEM((NN * NN,), jnp.float32),
            pltpu.VMEM((NTRI_PAD,), jnp.int32),
            pltpu.VMEM((NG,), jnp.float32),
            pltpu.VMEM((ORW + 16,), jnp.float32),
            pltpu.SemaphoreType.DMA,
            pltpu.SemaphoreType.DMA,
        ],
    )
    def extract_kernel(dots_hbm, exit_hbm, tri_hbm, out_hbm,
                       dbuf_a, dbuf_b, tri_v, exit_v, obuf, sem_a, sem_b):
        c = lax.axis_index("c")
        s = lax.axis_index("s")
        w = s * NC + c
        iota16 = lax.iota(jnp.int32, 16)
        dbufs = (dbuf_a, dbuf_b)
        sems = (sem_a, sem_b)

        pltpu.sync_copy(tri_hbm, tri_v)
        pltpu.sync_copy(exit_hbm, exit_v)

        def stage(g):
            gg = w * GPT + g
            return pltpu.async_copy(dots_hbm.at[pl.ds(gg * NN * NN, NN * NN)],
                                    dbufs[g % 2], sems[g % 2])

        stage_d = [None, None]
        stage_d[0] = stage(0)
        for g in range(GPT):
            if g + 1 < GPT:
                stage_d[(g + 1) % 2] = stage(g + 1)
            stage_d[g % 2].wait()
            dbuf = dbufs[g % 2]

            @plsc.parallel_loop(0, NTRI_PAD // 16, unroll=8)
            def _(k):
                tv = tri_v[pl.ds(k * 16, 16)]
                obuf[pl.ds(g * OW + k * 16, 16)] = plsc.load_gather(dbuf, [tv])

        ev = plsc.load_gather(exit_v, [w * GPT + iota16])
        plsc.store_scatter(obuf, [iota16 * OW + (OW - 1)], ev)

        pltpu.sync_copy(obuf.at[pl.ds(0, ORW)],
                        out_hbm.at[pl.ds(w * ORW, ORW)])

    return extract_kernel(dots_flat, exit_flat, tri)


def _ln(x, g, b):
    mu = jnp.mean(x, axis=-1, keepdims=True)
    xm = x - mu
    var = jnp.mean(xm * xm, axis=-1, keepdims=True)
    return xm * lax.rsqrt(var + 1e-5) * g + b


def _tc_dense(C2, emb_pad, gin_W1, gin_b1, gin_ln_g, gin_ln_b, gin_W2, gin_b2,
              post_W1, post_b1, post_W2, post_b2, norm_g, norm_b,
              exit_W1, exit_b1, exit_ln_g, exit_ln_b, exit_w2row, exit_b2v):
    GB = 16
    ROWS = GB * NN

    def body(C_ref, emb_ref, W1_ref, b1_ref, lg_ref, lb_ref, W2_ref, b2_ref,
             pW1_ref, pb1_ref, pW2_ref, pb2_ref, ng_ref, nb_ref,
             eW1_ref, eb1_ref, eg_ref, ebb_ref, ew2_ref, eb2_ref,
             dots_ref, exit_ref, A1_ref):
        @pl.when(pl.program_id(0) == 0)
        def _():
            A1_ref[...] = jnp.dot(emb_ref[...], W1_ref[...],
                                  preferred_element_type=jnp.float32)
        def dot16(a, b):
            return jnp.dot(a.astype(jnp.bfloat16), b.astype(jnp.bfloat16),
                           preferred_element_type=jnp.float32)

        Cb = C_ref[...]
        h = dot16(Cb, A1_ref[...]) + b1_ref[...]
        h = jnp.maximum(_ln(h, lg_ref[...], lb_ref[...]), 0.0)
        h = dot16(h, W2_ref[...]) + b2_ref[...]
        h = jnp.maximum(dot16(h, pW1_ref[...]) + pb1_ref[...], 0.0)
        h = dot16(h, pW2_ref[...]) + pb2_ref[...]
        x = _ln(h, ng_ref[...], nb_ref[...])

        ridx = lax.broadcasted_iota(jnp.int32, (GB, ROWS), 1)
        gidx = lax.broadcasted_iota(jnp.int32, (GB, ROWS), 0)
        S = jnp.where(ridx // NN == gidx, jnp.float32(1.0 / NN), jnp.float32(0.0))
        means = jnp.dot(S, x, preferred_element_type=jnp.float32)

        e = jnp.dot(means, eW1_ref[...], preferred_element_type=jnp.float32) + eb1_ref[...]
        e = jnp.maximum(_ln(e, eg_ref[...], ebb_ref[...]), 0.0)
        ex = jnp.sum(e * ew2_ref[...], axis=-1, keepdims=True) + eb2_ref[0:1, 0:1]
        exit_ref[...] = ex

        x16 = x.astype(jnp.bfloat16)
        for g in range(GB):
            xg = lax.slice(x16, (g * NN, 0), ((g + 1) * NN, D))
            dg = lax.dot_general(xg, xg, (((1,), (1,)), ((), ())),
                                 preferred_element_type=jnp.float32)
            dots_ref[g, :, :] = dg

    wspec = pl.BlockSpec((D, D), lambda i: (0, 0))
    vspec = pl.BlockSpec((1, D), lambda i: (0, 0))
    return pl.pallas_call(
        body,
        grid=(NG // GB,),
        in_specs=[
            pl.BlockSpec((ROWS, D), lambda i: (i, 0)),
            wspec, wspec, vspec, vspec, vspec, wspec, vspec,
            wspec, vspec, wspec, vspec, vspec, vspec,
            wspec, vspec, vspec, vspec, vspec, vspec,
        ],
        out_specs=[
            pl.BlockSpec((GB, NN, NN), lambda i: (i, 0, 0)),
            pl.BlockSpec((GB, 1), lambda i: (i, 0)),
        ],
        out_shape=[
            jax.ShapeDtypeStruct((NG, NN, NN), jnp.float32),
            jax.ShapeDtypeStruct((NG, 1), jnp.float32),
        ],
        scratch_shapes=[pltpu.VMEM((D, D), jnp.float32)],
    )(C2, emb_pad, gin_W1, gin_b1, gin_ln_g, gin_ln_b, gin_W2, gin_b2,
      post_W1, post_b1, post_W2, post_b2, norm_g, norm_b,
      exit_W1, exit_b1, exit_ln_g, exit_ln_b, exit_w2row, exit_b2v)


def kernel(node_ids, edge_index, ptr, emb, gin_W1, gin_b1, gin_ln_g, gin_ln_b,
           gin_W2, gin_b2, post_W1, post_b1, post_W2, post_b2, norm_g, norm_b,
           exit_W1, exit_b1, exit_ln_g, exit_ln_b, exit_W2, exit_b2):
    nids = node_ids.reshape(NT)

    keys = _sc_keys(nids, edge_index)
    C_flat = _sc_histogram(keys)
    C2 = C_flat.reshape(NT, D)

    emb_pad = jnp.zeros((D, D), jnp.float32).at[:NN].set(emb)
    r = lambda v: v.reshape(1, D)
    dots3, exit_out = _tc_dense(
        C2, emb_pad, gin_W1, r(gin_b1), r(gin_ln_g), r(gin_ln_b), gin_W2,
        r(gin_b2), post_W1, r(post_b1), post_W2, r(post_b2), r(norm_g),
        r(norm_b), exit_W1, r(exit_b1), r(exit_ln_g), r(exit_ln_b),
        exit_W2.reshape(1, D), jnp.full((1, D), exit_b2[0], jnp.float32))

    out_flat = _sc_extract(dots3.reshape(NG * NN * NN), exit_out.reshape(NG),
                           jnp.asarray(TRI_IDX))
    return out_flat.reshape(NG, OW)


# speedup vs baseline: 1.0021x; 1.0021x over previous
import functools

import numpy as np
import jax
import jax.numpy as jnp
from jax import lax
from jax.experimental import pallas as pl
from jax.experimental.pallas import tpu as pltpu
from jax.experimental.pallas import tpu_sc as plsc

NG = 512
NN = 100
NT = NG * NN
NE = NT * 16
D = 128

NC, NS = 2, 16
NW = NC * NS
HALF = NT // NC
CH = HALF // 2
CHW = CH * D
TRASH = CHW
SH_W = CHW + 2048
PTW = CHW // NS
ZB = 10240
EB_A = 6400
ESHARD = NE // NW
NSHARD = NT // NW
NK = NE + NT
KSHARD = NK // NS
KB = 3200
KR = KB // 128
NKB = KSHARD // KB

_IU, _JU = np.triu_indices(NN, k=1)
NTRI = _IU.size
OW = NTRI + 1
NTRI_PAD = 4960
TRI_IDX = np.zeros(NTRI_PAD, dtype=np.int32)
TRI_IDX[:NTRI] = _IU * NN + _JU
GPT = NG // NW
ORW = GPT * OW


def _sc_keys(nids, edge_index):
    mesh = plsc.VectorSubcoreMesh(
        core_axis_name="c", subcore_axis_name="s", num_cores=NC, num_subcores=NS)

    @functools.partial(
        pl.kernel,
        out_type=jax.ShapeDtypeStruct((NK,), jnp.int32),
        mesh=mesh,
        compiler_params=pltpu.CompilerParams(needs_layout_passes=False),
        scratch_types=[
            pltpu.VMEM((NT,), jnp.int32),
            pltpu.VMEM((EB_A,), jnp.int32),
            pltpu.VMEM((EB_A,), jnp.int32),
            pltpu.VMEM((EB_A,), jnp.int32),
        ],
    )
    def keys_kernel(nids_hbm, edge_hbm, out_hbm, nids_v, sbuf, dbuf, kbuf):
        c = lax.axis_index("c")
        s = lax.axis_index("s")
        w = s * NC + c
        iota16 = lax.iota(jnp.int32, 16)

        pltpu.sync_copy(nids_hbm, nids_v)

        for b in range(ESHARD // EB_A):
            ebase = w * ESHARD + b * EB_A
            pltpu.sync_copy(edge_hbm.at[0, pl.ds(ebase, EB_A)], sbuf)
            pltpu.sync_copy(edge_hbm.at[1, pl.ds(ebase, EB_A)], dbuf)

            @plsc.parallel_loop(0, EB_A // 16, unroll=8)
            def _(v):
                sv = sbuf[pl.ds(v * 16, 16)]
                dv = dbuf[pl.ds(v * 16, 16)]
                nv = plsc.load_gather(nids_v, [sv])
                kbuf[pl.ds(v * 16, 16)] = dv * D + nv
            pltpu.sync_copy(kbuf, out_hbm.at[pl.ds(ebase, EB_A)])

        nbase = w * NSHARD

        @plsc.parallel_loop(0, NSHARD // 16, unroll=8)
        def _(v):
            iv = nbase + v * 16 + iota16
            nv = plsc.load_gather(nids_v, [iv])
            kbuf[pl.ds(v * 16, 16)] = iv * D + nv
        pltpu.sync_copy(kbuf.at[pl.ds(0, NSHARD)],
                        out_hbm.at[pl.ds(NE + nbase, NSHARD)])

    return keys_kernel(nids, edge_index)


def _sc_histogram(keys):
    mesh = plsc.VectorSubcoreMesh(
        core_axis_name="c", subcore_axis_name="s", num_cores=NC, num_subcores=NS)

    @functools.partial(
        pl.kernel,
        out_type=jax.ShapeDtypeStruct((NT * D,), jnp.float32),
        mesh=mesh,
        compiler_params=pltpu.CompilerParams(needs_layout_passes=False),
        scratch_types=[
            pltpu.VMEM((KB,), jnp.int32),
            pltpu.VMEM((KB,), jnp.int32),
            pltpu.VMEM((KB,), jnp.int32),
            pltpu.VMEM((KB,), jnp.float32),
            pltpu.VMEM((ZB,), jnp.float32),
            pltpu.VMEM_SHARED((SH_W,), jnp.float32),
            pltpu.SemaphoreType.DMA,
        ],
    )
    def hist_kernel(keys_hbm, out_hbm, kbuf, idx_a, idx_b, ones1d, zeros_v,
                    shared, sem):
        c = lax.axis_index("c")
        s = lax.axis_index("s")

        def ones_body(v, _):
            ones1d[pl.ds(v * 16, 16)] = jnp.ones((16,), jnp.float32)
            return 0
        lax.fori_loop(0, KB // 16, ones_body, 0)

        def zbuf_body(i, _):
            zeros_v[pl.ds(i * 16, 16)] = jnp.zeros((16,), jnp.float32)
            return 0
        lax.fori_loop(0, ZB // 16, zbuf_body, 0)

        for chunk in range(2):
            lo = (c * HALF + chunk * CH) * D
            for z in range(PTW // ZB):
                pltpu.sync_copy(zeros_v,
                                shared.at[pl.ds(s * PTW + z * ZB, ZB)])
            plsc.subcore_barrier()

            descs = [None, None]
            for b in range(NKB):
                kbase = s * KSHARD + b * KB
                pltpu.sync_copy(keys_hbm.at[pl.ds(kbase, KB)], kbuf)
                ib = idx_a if b % 2 == 0 else idx_b
                if descs[b % 2] is not None:
                    descs[b % 2].wait()

                @plsc.parallel_loop(0, KB // 16, unroll=8)
                def _(v):
                    kv = kbuf[pl.ds(v * 16, 16)]
                    rel = kv - lo
                    ok = (rel >= 0) & (rel < CHW)
                    ib[pl.ds(v * 16, 16)] = jnp.where(ok, rel, -1)
                descs[b % 2] = pltpu.async_copy(
                    ones1d,
                    shared.at[plsc.Indices(ib, ignored_value=-1)],
                    sem, add=True)
            for dsc in descs:
                if dsc is not None:
                    dsc.wait()

            plsc.subcore_barrier()
            pltpu.sync_copy(shared.at[pl.ds(s * PTW, PTW)],
                            out_hbm.at[pl.ds(lo + s * PTW, PTW)])

    return hist_kernel(keys)


def _sc_extract(dots_flat, exit_flat, tri):
    mesh = plsc.VectorSubcoreMesh(
        core_axis_name="c", subcore_axis_name="s", num_cores=NC, num_subcores=NS)

    @functools.partial(
        pl.kernel,
        out_type=jax.ShapeDtypeStruct((NG * OW,), jnp.float32),
        mesh=mesh,
        compiler_params=pltpu.CompilerParams(needs_layout_passes=False),
        scratch_types=[
            pltpu.VMEM((NN * NN,), jnp.float32),
            pltpu.VMEM((NN * NN,), jnp.float32),
            pltpu.VMEM((NTRI_PAD,), jnp.int32),
            pltpu.VMEM((NG,), jnp.float32),
            pltpu.VMEM((ORW + 16,), jnp.float32),
            pltpu.SemaphoreType.DMA,
            pltpu.SemaphoreType.DMA,
        ],
    )
    def extract_kernel(dots_hbm, exit_hbm, tri_hbm, out_hbm,
                       dbuf_a, dbuf_b, tri_v, exit_v, obuf, sem_a, sem_b):
        c = lax.axis_index("c")
        s = lax.axis_index("s")
        w = s * NC + c
        iota16 = lax.iota(jnp.int32, 16)
        dbufs = (dbuf_a, dbuf_b)
        sems = (sem_a, sem_b)

        pltpu.sync_copy(tri_hbm, tri_v)
        pltpu.sync_copy(exit_hbm, exit_v)

        def stage(g):
            gg = w * GPT + g
            return pltpu.async_copy(dots_hbm.at[pl.ds(gg * NN * NN, NN * NN)],
                                    dbufs[g % 2], sems[g % 2])

        stage_d = [None, None]
        stage_d[0] = stage(0)
        for g in range(GPT):
            if g + 1 < GPT:
                stage_d[(g + 1) % 2] = stage(g + 1)
            stage_d[g % 2].wait()
            dbuf = dbufs[g % 2]

            @plsc.parallel_loop(0, NTRI_PAD // 16, unroll=8)
            def _(k):
                tv = tri_v[pl.ds(k * 16, 16)]
                obuf[pl.ds(g * OW + k * 16, 16)] = plsc.load_gather(dbuf, [tv])

        ev = plsc.load_gather(exit_v, [w * GPT + iota16])
        plsc.store_scatter(obuf, [iota16 * OW + (OW - 1)], ev)

        pltpu.sync_copy(obuf.at[pl.ds(0, ORW)],
                        out_hbm.at[pl.ds(w * ORW, ORW)])

    return extract_kernel(dots_flat, exit_flat, tri)


def _ln(x, g, b):
    mu = jnp.mean(x, axis=-1, keepdims=True)
    xm = x - mu
    var = jnp.mean(xm * xm, axis=-1, keepdims=True)
    return xm * lax.rsqrt(var + 1e-5) * g + b


def _tc_dense(C2, emb_pad, gin_W1, gin_b1, gin_ln_g, gin_ln_b, gin_W2, gin_b2,
              post_W1, post_b1, post_W2, post_b2, norm_g, norm_b,
              exit_W1, exit_b1, exit_ln_g, exit_ln_b, exit_w2row, exit_b2v):
    GB = 16
    ROWS = GB * NN

    def body(C_ref, emb_ref, W1_ref, b1_ref, lg_ref, lb_ref, W2_ref, b2_ref,
             pW1_ref, pb1_ref, pW2_ref, pb2_ref, ng_ref, nb_ref,
             eW1_ref, eb1_ref, eg_ref, ebb_ref, ew2_ref, eb2_ref,
             dots_ref, exit_ref, A1_ref):
        @pl.when(pl.program_id(0) == 0)
        def _():
            A1_ref[...] = jnp.dot(emb_ref[...], W1_ref[...],
                                  preferred_element_type=jnp.float32)
        def dotf(a, b):
            return jnp.dot(a, b, preferred_element_type=jnp.float32)

        Cb = C_ref[...]
        h = dotf(Cb, A1_ref[...]) + b1_ref[...]
        h = jnp.maximum(_ln(h, lg_ref[...], lb_ref[...]), 0.0)
        h = dotf(h, W2_ref[...]) + b2_ref[...]
        h = jnp.maximum(dotf(h, pW1_ref[...]) + pb1_ref[...], 0.0)
        h = dotf(h, pW2_ref[...]) + pb2_ref[...]
        x = _ln(h, ng_ref[...], nb_ref[...])

        ridx = lax.broadcasted_iota(jnp.int32, (GB, ROWS), 1)
        gidx = lax.broadcasted_iota(jnp.int32, (GB, ROWS), 0)
        S = jnp.where(ridx // NN == gidx, jnp.float32(1.0 / NN), jnp.float32(0.0))
        means = jnp.dot(S, x, preferred_element_type=jnp.float32)

        e = jnp.dot(means, eW1_ref[...], preferred_element_type=jnp.float32) + eb1_ref[...]
        e = jnp.maximum(_ln(e, eg_ref[...], ebb_ref[...]), 0.0)
        ex = jnp.sum(e * ew2_ref[...], axis=-1, keepdims=True) + eb2_ref[0:1, 0:1]
        exit_ref[...] = ex

        for g in range(GB):
            xg = lax.slice(x, (g * NN, 0), ((g + 1) * NN, D))
            dg = lax.dot_general(xg, xg, (((1,), (1,)), ((), ())),
                                 preferred_element_type=jnp.float32)
            dots_ref[g, :, :] = dg

    wspec = pl.BlockSpec((D, D), lambda i: (0, 0))
    vspec = pl.BlockSpec((1, D), lambda i: (0, 0))
    return pl.pallas_call(
        body,
        grid=(NG // GB,),
        in_specs=[
            pl.BlockSpec((ROWS, D), lambda i: (i, 0)),
            wspec, wspec, vspec, vspec, vspec, wspec, vspec,
            wspec, vspec, wspec, vspec, vspec, vspec,
            wspec, vspec, vspec, vspec, vspec, vspec,
        ],
        out_specs=[
            pl.BlockSpec((GB, NN, NN), lambda i: (i, 0, 0)),
            pl.BlockSpec((GB, 1), lambda i: (i, 0)),
        ],
        out_shape=[
            jax.ShapeDtypeStruct((NG, NN, NN), jnp.float32),
            jax.ShapeDtypeStruct((NG, 1), jnp.float32),
        ],
        scratch_shapes=[pltpu.VMEM((D, D), jnp.float32)],
    )(C2, emb_pad, gin_W1, gin_b1, gin_ln_g, gin_ln_b, gin_W2, gin_b2,
      post_W1, post_b1, post_W2, post_b2, norm_g, norm_b,
      exit_W1, exit_b1, exit_ln_g, exit_ln_b, exit_w2row, exit_b2v)


def kernel(node_ids, edge_index, ptr, emb, gin_W1, gin_b1, gin_ln_g, gin_ln_b,
           gin_W2, gin_b2, post_W1, post_b1, post_W2, post_b2, norm_g, norm_b,
           exit_W1, exit_b1, exit_ln_g, exit_ln_b, exit_W2, exit_b2):
    nids = node_ids.reshape(NT)

    keys = _sc_keys(nids, edge_index)
    C_flat = _sc_histogram(keys)
    C2 = C_flat.reshape(NT, D)

    emb_pad = jnp.zeros((D, D), jnp.float32).at[:NN].set(emb)
    r = lambda v: v.reshape(1, D)
    dots3, exit_out = _tc_dense(
        C2, emb_pad, gin_W1, r(gin_b1), r(gin_ln_g), r(gin_ln_b), gin_W2,
        r(gin_b2), post_W1, r(post_b1), post_W2, r(post_b2), r(norm_g),
        r(norm_b), exit_W1, r(exit_b1), r(exit_ln_g), r(exit_ln_b),
        exit_W2.reshape(1, D), jnp.full((1, D), exit_b2[0], jnp.float32))

    out_flat = _sc_extract(dots3.reshape(NG * NN * NN), exit_out.reshape(NG),
                           jnp.asarray(TRI_IDX))
    return out_flat.reshape(NG, OW)


# speedup vs baseline: 1.1259x; 1.1236x over previous
import functools

import numpy as np
import jax
import jax.numpy as jnp
from jax import lax
from jax.experimental import pallas as pl
from jax.experimental.pallas import tpu as pltpu
from jax.experimental.pallas import tpu_sc as plsc

NG = 512
NN = 100
NT = NG * NN
NE = NT * 16
D = 128

NC, NS = 2, 16
NW = NC * NS
HALF = NT // NC
CH = HALF // 2
CHW = CH * D
TRASH = CHW
SH_W = CHW + 2048
PTW = CHW // NS
ZB = 10240
EB_A = 6400
ESHARD = NE // NW
NSHARD = NT // NW
NK = NE + NT
KSHARD = NK // NS
KB = 3200
KR = KB // 128
NKB = KSHARD // KB

_IU, _JU = np.triu_indices(NN, k=1)
NTRI = _IU.size
OW = NTRI + 1
NTRI_PAD = 4960
TRI_IDX = np.zeros(NTRI_PAD, dtype=np.int32)
TRI_IDX[:NTRI] = _IU * D + _JU
GPT = NG // NW
ORW = GPT * OW


def _sc_keys(nids, edge_index):
    mesh = plsc.VectorSubcoreMesh(
        core_axis_name="c", subcore_axis_name="s", num_cores=NC, num_subcores=NS)

    @functools.partial(
        pl.kernel,
        out_type=jax.ShapeDtypeStruct((NK,), jnp.int32),
        mesh=mesh,
        compiler_params=pltpu.CompilerParams(needs_layout_passes=False),
        scratch_types=[
            pltpu.VMEM((NT,), jnp.int32),
            pltpu.VMEM((EB_A,), jnp.int32),
            pltpu.VMEM((EB_A,), jnp.int32),
            pltpu.VMEM((EB_A,), jnp.int32),
        ],
    )
    def keys_kernel(nids_hbm, edge_hbm, out_hbm, nids_v, sbuf, dbuf, kbuf):
        c = lax.axis_index("c")
        s = lax.axis_index("s")
        w = s * NC + c
        iota16 = lax.iota(jnp.int32, 16)

        pltpu.sync_copy(nids_hbm, nids_v)

        for b in range(ESHARD // EB_A):
            ebase = w * ESHARD + b * EB_A
            pltpu.sync_copy(edge_hbm.at[0, pl.ds(ebase, EB_A)], sbuf)
            pltpu.sync_copy(edge_hbm.at[1, pl.ds(ebase, EB_A)], dbuf)

            @plsc.parallel_loop(0, EB_A // 16, unroll=8)
            def _(v):
                sv = sbuf[pl.ds(v * 16, 16)]
                dv = dbuf[pl.ds(v * 16, 16)]
                nv = plsc.load_gather(nids_v, [sv])
                kbuf[pl.ds(v * 16, 16)] = dv * D + nv
            pltpu.sync_copy(kbuf, out_hbm.at[pl.ds(ebase, EB_A)])

        nbase = w * NSHARD

        @plsc.parallel_loop(0, NSHARD // 16, unroll=8)
        def _(v):
            iv = nbase + v * 16 + iota16
            nv = plsc.load_gather(nids_v, [iv])
            kbuf[pl.ds(v * 16, 16)] = iv * D + nv
        pltpu.sync_copy(kbuf.at[pl.ds(0, NSHARD)],
                        out_hbm.at[pl.ds(NE + nbase, NSHARD)])

    return keys_kernel(nids, edge_index)


def _sc_histogram(keys):
    mesh = plsc.VectorSubcoreMesh(
        core_axis_name="c", subcore_axis_name="s", num_cores=NC, num_subcores=NS)

    @functools.partial(
        pl.kernel,
        out_type=jax.ShapeDtypeStruct((NT * D,), jnp.float32),
        mesh=mesh,
        compiler_params=pltpu.CompilerParams(needs_layout_passes=False),
        scratch_types=[
            pltpu.VMEM((KB,), jnp.int32),
            pltpu.VMEM((KB,), jnp.int32),
            pltpu.VMEM((KB,), jnp.int32),
            pltpu.VMEM((KB,), jnp.float32),
            pltpu.VMEM((ZB,), jnp.float32),
            pltpu.VMEM_SHARED((SH_W,), jnp.float32),
            pltpu.SemaphoreType.DMA,
            pltpu.SemaphoreType.DMA,
        ],
    )
    def hist_kernel(keys_hbm, out_hbm, kbuf, idx_a, idx_b, ones1d, zeros_v,
                    shared, sem, sem_io):
        c = lax.axis_index("c")
        s = lax.axis_index("s")

        def ones_body(v, _):
            ones1d[pl.ds(v * 16, 16)] = jnp.ones((16,), jnp.float32)
            return 0
        lax.fori_loop(0, KB // 16, ones_body, 0)

        def zbuf_body(i, _):
            zeros_v[pl.ds(i * 16, 16)] = jnp.zeros((16,), jnp.float32)
            return 0
        lax.fori_loop(0, ZB // 16, zbuf_body, 0)

        dump_d = None
        for chunk in range(2):
            lo = (c * HALF + chunk * CH) * D
            if dump_d is not None:
                dump_d.wait()
            zds = [pltpu.async_copy(zeros_v,
                                    shared.at[pl.ds(s * PTW + z * ZB, ZB)],
                                    sem_io)
                   for z in range(PTW // ZB)]
            for zd in zds:
                zd.wait()
            plsc.subcore_barrier()

            descs = [None, None]
            for b in range(NKB):
                kbase = s * KSHARD + b * KB
                pltpu.sync_copy(keys_hbm.at[pl.ds(kbase, KB)], kbuf)
                ib = idx_a if b % 2 == 0 else idx_b
                if descs[b % 2] is not None:
                    descs[b % 2].wait()

                @plsc.parallel_loop(0, KB // 16, unroll=8)
                def _(v):
                    kv = kbuf[pl.ds(v * 16, 16)]
                    rel = kv - lo
                    ok = (rel >= 0) & (rel < CHW)
                    ib[pl.ds(v * 16, 16)] = jnp.where(ok, rel, -1)
                descs[b % 2] = pltpu.async_copy(
                    ones1d,
                    shared.at[plsc.Indices(ib, ignored_value=-1)],
                    sem, add=True)
            for dsc in descs:
                if dsc is not None:
                    dsc.wait()

            plsc.subcore_barrier()
            dump_d = pltpu.async_copy(shared.at[pl.ds(s * PTW, PTW)],
                                      out_hbm.at[pl.ds(lo + s * PTW, PTW)],
                                      sem_io)
        dump_d.wait()

    return hist_kernel(keys)


def _sc_extract(dots_flat, exit_flat, tri):
    mesh = plsc.VectorSubcoreMesh(
        core_axis_name="c", subcore_axis_name="s", num_cores=NC, num_subcores=NS)

    @functools.partial(
        pl.kernel,
        out_type=jax.ShapeDtypeStruct((NG * OW,), jnp.float32),
        mesh=mesh,
        compiler_params=pltpu.CompilerParams(needs_layout_passes=False),
        scratch_types=[
            pltpu.VMEM((NN * D,), jnp.float32),
            pltpu.VMEM((NN * D,), jnp.float32),
            pltpu.VMEM((NTRI_PAD,), jnp.int32),
            pltpu.VMEM((NG,), jnp.float32),
            pltpu.VMEM((ORW + 16,), jnp.float32),
            pltpu.SemaphoreType.DMA,
            pltpu.SemaphoreType.DMA,
        ],
    )
    def extract_kernel(dots_hbm, exit_hbm, tri_hbm, out_hbm,
                       dbuf_a, dbuf_b, tri_v, exit_v, obuf, sem_a, sem_b):
        c = lax.axis_index("c")
        s = lax.axis_index("s")
        w = s * NC + c
        iota16 = lax.iota(jnp.int32, 16)
        dbufs = (dbuf_a, dbuf_b)
        sems = (sem_a, sem_b)

        pltpu.sync_copy(tri_hbm, tri_v)
        pltpu.sync_copy(exit_hbm, exit_v)

        def stage(g):
            gg = w * GPT + g
            return pltpu.async_copy(dots_hbm.at[pl.ds(gg * NN * D, NN * D)],
                                    dbufs[g % 2], sems[g % 2])

        stage_d = [None, None]
        stage_d[0] = stage(0)
        for g in range(GPT):
            if g + 1 < GPT:
                stage_d[(g + 1) % 2] = stage(g + 1)
            stage_d[g % 2].wait()
            dbuf = dbufs[g % 2]

            @plsc.parallel_loop(0, NTRI_PAD // 16, unroll=8)
            def _(k):
                tv = tri_v[pl.ds(k * 16, 16)]
                obuf[pl.ds(g * OW + k * 16, 16)] = plsc.load_gather(dbuf, [tv])

        ev = plsc.load_gather(exit_v, [w * GPT + iota16])
        plsc.store_scatter(obuf, [iota16 * OW + (OW - 1)], ev)

        pltpu.sync_copy(obuf.at[pl.ds(0, ORW)],
                        out_hbm.at[pl.ds(w * ORW, ORW)])

    return extract_kernel(dots_flat, exit_flat, tri)


def _ln(x, g, b):
    mu = jnp.mean(x, axis=-1, keepdims=True)
    xm = x - mu
    var = jnp.mean(xm * xm, axis=-1, keepdims=True)
    return xm * lax.rsqrt(var + 1e-5) * g + b


def _tc_dense(C2, emb_pad, gin_W1, gin_b1, gin_ln_g, gin_ln_b, gin_W2, gin_b2,
              post_W1, post_b1, post_W2, post_b2, norm_g, norm_b,
              exit_W1, exit_b1, exit_ln_g, exit_ln_b, exit_w2row, exit_b2v):
    GB = 16
    ROWS = GB * NN

    def body(C_ref, emb_ref, W1_ref, b1_ref, lg_ref, lb_ref, W2_ref, b2_ref,
             pW1_ref, pb1_ref, pW2_ref, pb2_ref, ng_ref, nb_ref,
             eW1_ref, eb1_ref, eg_ref, ebb_ref, ew2_ref, eb2_ref,
             dots_ref, exit_ref, A1_ref):
        @pl.when(pl.program_id(0) == 0)
        def _():
            A1_ref[...] = jnp.dot(emb_ref[...], W1_ref[...],
                                  preferred_element_type=jnp.float32)
        def dotf(a, b):
            return jnp.dot(a, b, preferred_element_type=jnp.float32)

        Cb = C_ref[...]
        h = dotf(Cb, A1_ref[...]) + b1_ref[...]
        h = jnp.maximum(_ln(h, lg_ref[...], lb_ref[...]), 0.0)
        h = dotf(h, W2_ref[...]) + b2_ref[...]
        h = jnp.maximum(dotf(h, pW1_ref[...]) + pb1_ref[...], 0.0)
        h = dotf(h, pW2_ref[...]) + pb2_ref[...]
        x = _ln(h, ng_ref[...], nb_ref[...])

        ridx = lax.broadcasted_iota(jnp.int32, (GB, ROWS), 1)
        gidx = lax.broadcasted_iota(jnp.int32, (GB, ROWS), 0)
        S = jnp.where(ridx // NN == gidx, jnp.float32(1.0 / NN), jnp.float32(0.0))
        means = jnp.dot(S, x, preferred_element_type=jnp.float32)

        e = jnp.dot(means, eW1_ref[...], preferred_element_type=jnp.float32) + eb1_ref[...]
        e = jnp.maximum(_ln(e, eg_ref[...], ebb_ref[...]), 0.0)
        ex = jnp.sum(e * ew2_ref[...], axis=-1, keepdims=True) + eb2_ref[0:1, 0:1]
        exit_ref[...] = ex

        for g in range(GB):
            xg = lax.slice(x, (g * NN, 0), ((g + 1) * NN, D))
            xgp = jnp.concatenate([xg, jnp.zeros((D - NN, D), jnp.float32)], 0)
            dg = lax.dot_general(xg, xgp, (((1,), (1,)), ((), ())),
                                 preferred_element_type=jnp.float32)
            dots_ref[pl.ds(g * NN, NN), :] = dg

    wspec = pl.BlockSpec((D, D), lambda i: (0, 0))
    vspec = pl.BlockSpec((1, D), lambda i: (0, 0))
    return pl.pallas_call(
        body,
        grid=(NG // GB,),
        in_specs=[
            pl.BlockSpec((ROWS, D), lambda i: (i, 0)),
            wspec, wspec, vspec, vspec, vspec, wspec, vspec,
            wspec, vspec, wspec, vspec, vspec, vspec,
            wspec, vspec, vspec, vspec, vspec, vspec,
        ],
        out_specs=[
            pl.BlockSpec((GB * NN, D), lambda i: (i, 0)),
            pl.BlockSpec((GB, 1), lambda i: (i, 0)),
        ],
        out_shape=[
            jax.ShapeDtypeStruct((NT, D), jnp.float32),
            jax.ShapeDtypeStruct((NG, 1), jnp.float32),
        ],
        scratch_shapes=[pltpu.VMEM((D, D), jnp.float32)],
    )(C2, emb_pad, gin_W1, gin_b1, gin_ln_g, gin_ln_b, gin_W2, gin_b2,
      post_W1, post_b1, post_W2, post_b2, norm_g, norm_b,
      exit_W1, exit_b1, exit_ln_g, exit_ln_b, exit_w2row, exit_b2v)


def kernel(node_ids, edge_index, ptr, emb, gin_W1, gin_b1, gin_ln_g, gin_ln_b,
           gin_W2, gin_b2, post_W1, post_b1, post_W2, post_b2, norm_g, norm_b,
           exit_W1, exit_b1, exit_ln_g, exit_ln_b, exit_W2, exit_b2):
    nids = node_ids.reshape(NT)

    keys = _sc_keys(nids, edge_index)
    C_flat = _sc_histogram(keys)
    C2 = C_flat.reshape(NT, D)

    emb_pad = jnp.zeros((D, D), jnp.float32).at[:NN].set(emb)
    r = lambda v: v.reshape(1, D)
    dots3, exit_out = _tc_dense(
        C2, emb_pad, gin_W1, r(gin_b1), r(gin_ln_g), r(gin_ln_b), gin_W2,
        r(gin_b2), post_W1, r(post_b1), post_W2, r(post_b2), r(norm_g),
        r(norm_b), exit_W1, r(exit_b1), r(exit_ln_g), r(exit_ln_b),
        exit_W2.reshape(1, D), jnp.full((1, D), exit_b2[0], jnp.float32))

    out_flat = _sc_extract(dots3.reshape(NT * D), exit_out.reshape(NG),
                           jnp.asarray(TRI_IDX))
    return out_flat.reshape(NG, OW)


# speedup vs baseline: 1.1590x; 1.0294x over previous
import functools

import numpy as np
import jax
import jax.numpy as jnp
from jax import lax
from jax.experimental import pallas as pl
from jax.experimental.pallas import tpu as pltpu
from jax.experimental.pallas import tpu_sc as plsc

NG = 512
NN = 100
NT = NG * NN
NE = NT * 16
D = 128

NC, NS = 2, 16
NW = NC * NS
HALF = NT // NC
CH = HALF // 2
CHW = CH * D
TRASH = CHW
SH_W = CHW + 2048
PTW = CHW // NS
ZB = 10240
EB_A = 6400
ESHARD = NE // NW
NSHARD = NT // NW
NK = NE + NT
KSHARD = NK // NS
KB = 3200
KR = KB // 128
NKB = KSHARD // KB

_IU, _JU = np.triu_indices(NN, k=1)
NTRI = _IU.size
OW = NTRI + 1
NTRI_PAD = 4960
TRI_IDX = np.zeros(NTRI_PAD, dtype=np.int32)
TRI_IDX[:NTRI] = _IU * D + _JU
GPT = NG // NW
ORW = GPT * OW


def _sc_keys(nids, edge_index):
    mesh = plsc.VectorSubcoreMesh(
        core_axis_name="c", subcore_axis_name="s", num_cores=NC, num_subcores=NS)

    @functools.partial(
        pl.kernel,
        out_type=jax.ShapeDtypeStruct((NK,), jnp.int32),
        mesh=mesh,
        compiler_params=pltpu.CompilerParams(needs_layout_passes=False),
        scratch_types=[
            pltpu.VMEM((NT,), jnp.int32),
            pltpu.VMEM((EB_A,), jnp.int32),
            pltpu.VMEM((EB_A,), jnp.int32),
            pltpu.VMEM((EB_A,), jnp.int32),
            pltpu.VMEM((EB_A,), jnp.int32),
            pltpu.VMEM((EB_A,), jnp.int32),
            pltpu.VMEM((EB_A,), jnp.int32),
            pltpu.SemaphoreType.DMA,
            pltpu.SemaphoreType.DMA,
        ],
    )
    def keys_kernel(nids_hbm, edge_hbm, out_hbm, nids_v, sbuf_a, sbuf_b,
                    dbuf_a, dbuf_b, kbuf_a, kbuf_b, sem_in, sem_out):
        c = lax.axis_index("c")
        s = lax.axis_index("s")
        w = s * NC + c
        iota16 = lax.iota(jnp.int32, 16)
        sbufs = (sbuf_a, sbuf_b)
        dbufs = (dbuf_a, dbuf_b)
        kbufs = (kbuf_a, kbuf_b)

        pltpu.sync_copy(nids_hbm, nids_v)

        NB_A = ESHARD // EB_A

        def eload(b):
            ebase = w * ESHARD + b * EB_A
            return (pltpu.async_copy(edge_hbm.at[0, pl.ds(ebase, EB_A)],
                                     sbufs[b % 2], sem_in),
                    pltpu.async_copy(edge_hbm.at[1, pl.ds(ebase, EB_A)],
                                     dbufs[b % 2], sem_in))

        lds = [None, None]
        sts = [None, None]
        lds[0] = eload(0)
        for b in range(NB_A):
            if b + 1 < NB_A:
                lds[(b + 1) % 2] = eload(b + 1)
            for dsc in lds[b % 2]:
                dsc.wait()
            if sts[b % 2] is not None:
                sts[b % 2].wait()
            sb, db, kb = sbufs[b % 2], dbufs[b % 2], kbufs[b % 2]

            @plsc.parallel_loop(0, EB_A // 16, unroll=8)
            def _(v):
                sv = sb[pl.ds(v * 16, 16)]
                dv = db[pl.ds(v * 16, 16)]
                nv = plsc.load_gather(nids_v, [sv])
                kb[pl.ds(v * 16, 16)] = dv * D + nv
            ebase = w * ESHARD + b * EB_A
            sts[b % 2] = pltpu.async_copy(
                kb, out_hbm.at[pl.ds(ebase, EB_A)], sem_out)

        nbase = w * NSHARD
        kb = kbufs[NB_A % 2]
        if sts[NB_A % 2] is not None:
            sts[NB_A % 2].wait()

        @plsc.parallel_loop(0, NSHARD // 16, unroll=8)
        def _(v):
            iv = nbase + v * 16 + iota16
            nv = plsc.load_gather(nids_v, [iv])
            kb[pl.ds(v * 16, 16)] = iv * D + nv
        pltpu.sync_copy(kb.at[pl.ds(0, NSHARD)],
                        out_hbm.at[pl.ds(NE + nbase, NSHARD)])
        for st in sts:
            if st is not None and st is not sts[NB_A % 2]:
                st.wait()

    return keys_kernel(nids, edge_index)


def _sc_histogram(keys):
    mesh = plsc.VectorSubcoreMesh(
        core_axis_name="c", subcore_axis_name="s", num_cores=NC, num_subcores=NS)

    @functools.partial(
        pl.kernel,
        out_type=jax.ShapeDtypeStruct((NT * D,), jnp.float32),
        mesh=mesh,
        compiler_params=pltpu.CompilerParams(needs_layout_passes=False),
        scratch_types=[
            pltpu.VMEM((KB,), jnp.int32),
            pltpu.VMEM((KB,), jnp.int32),
            pltpu.VMEM((KB,), jnp.int32),
            pltpu.VMEM((KB,), jnp.int32),
            pltpu.VMEM((KB,), jnp.float32),
            pltpu.VMEM((ZB,), jnp.float32),
            pltpu.VMEM_SHARED((SH_W,), jnp.float32),
            pltpu.SemaphoreType.DMA,
            pltpu.SemaphoreType.DMA,
            pltpu.SemaphoreType.DMA,
        ],
    )
    def hist_kernel(keys_hbm, out_hbm, kbuf_a, kbuf_b, idx_a, idx_b, ones1d,
                    zeros_v, shared, sem, sem_io, sem_k):
        c = lax.axis_index("c")
        s = lax.axis_index("s")

        def ones_body(v, _):
            ones1d[pl.ds(v * 16, 16)] = jnp.ones((16,), jnp.float32)
            return 0
        lax.fori_loop(0, KB // 16, ones_body, 0)

        def zbuf_body(i, _):
            zeros_v[pl.ds(i * 16, 16)] = jnp.zeros((16,), jnp.float32)
            return 0
        lax.fori_loop(0, ZB // 16, zbuf_body, 0)

        dump_d = None
        for chunk in range(2):
            lo = (c * HALF + chunk * CH) * D
            if dump_d is not None:
                dump_d.wait()
            zds = [pltpu.async_copy(zeros_v,
                                    shared.at[pl.ds(s * PTW + z * ZB, ZB)],
                                    sem_io)
                   for z in range(PTW // ZB)]
            for zd in zds:
                zd.wait()
            plsc.subcore_barrier()

            def kload(b):
                kbase = s * KSHARD + b * KB
                return pltpu.async_copy(keys_hbm.at[pl.ds(kbase, KB)],
                                        kbuf_a if b % 2 == 0 else kbuf_b,
                                        sem_k)

            descs = [None, None]
            lds = [None, None]
            lds[0] = kload(0)
            for b in range(NKB):
                if b + 1 < NKB:
                    lds[(b + 1) % 2] = kload(b + 1)
                lds[b % 2].wait()
                kb = kbuf_a if b % 2 == 0 else kbuf_b
                ib = idx_a if b % 2 == 0 else idx_b
                if descs[b % 2] is not None:
                    descs[b % 2].wait()

                @plsc.parallel_loop(0, KB // 16, unroll=8)
                def _(v):
                    kv = kb[pl.ds(v * 16, 16)]
                    rel = kv - lo
                    ok = (rel >= 0) & (rel < CHW)
                    ib[pl.ds(v * 16, 16)] = jnp.where(ok, rel, -1)
                descs[b % 2] = pltpu.async_copy(
                    ones1d,
                    shared.at[plsc.Indices(ib, ignored_value=-1)],
                    sem, add=True)
            for dsc in descs:
                if dsc is not None:
                    dsc.wait()

            plsc.subcore_barrier()
            dump_d = pltpu.async_copy(shared.at[pl.ds(s * PTW, PTW)],
                                      out_hbm.at[pl.ds(lo + s * PTW, PTW)],
                                      sem_io)
        dump_d.wait()

    return hist_kernel(keys)


def _sc_extract(dots_flat, exit_flat, tri):
    mesh = plsc.VectorSubcoreMesh(
        core_axis_name="c", subcore_axis_name="s", num_cores=NC, num_subcores=NS)

    @functools.partial(
        pl.kernel,
        out_type=jax.ShapeDtypeStruct((NG * OW,), jnp.float32),
        mesh=mesh,
        compiler_params=pltpu.CompilerParams(needs_layout_passes=False),
        scratch_types=[
            pltpu.VMEM((NN * D,), jnp.float32),
            pltpu.VMEM((NN * D,), jnp.float32),
            pltpu.VMEM((NTRI_PAD,), jnp.int32),
            pltpu.VMEM((NG,), jnp.float32),
            pltpu.VMEM((ORW + 16,), jnp.float32),
            pltpu.SemaphoreType.DMA,
            pltpu.SemaphoreType.DMA,
        ],
    )
    def extract_kernel(dots_hbm, exit_hbm, tri_hbm, out_hbm,
                       dbuf_a, dbuf_b, tri_v, exit_v, obuf, sem_a, sem_b):
        c = lax.axis_index("c")
        s = lax.axis_index("s")
        w = s * NC + c
        iota16 = lax.iota(jnp.int32, 16)
        dbufs = (dbuf_a, dbuf_b)
        sems = (sem_a, sem_b)

        pltpu.sync_copy(tri_hbm, tri_v)
        pltpu.sync_copy(exit_hbm, exit_v)

        def stage(g):
            gg = w * GPT + g
            return pltpu.async_copy(dots_hbm.at[pl.ds(gg * NN * D, NN * D)],
                                    dbufs[g % 2], sems[g % 2])

        stage_d = [None, None]
        stage_d[0] = stage(0)
        for g in range(GPT):
            if g + 1 < GPT:
                stage_d[(g + 1) % 2] = stage(g + 1)
            stage_d[g % 2].wait()
            dbuf = dbufs[g % 2]

            @plsc.parallel_loop(0, NTRI_PAD // 16, unroll=8)
            def _(k):
                tv = tri_v[pl.ds(k * 16, 16)]
                obuf[pl.ds(g * OW + k * 16, 16)] = plsc.load_gather(dbuf, [tv])

        ev = plsc.load_gather(exit_v, [w * GPT + iota16])
        plsc.store_scatter(obuf, [iota16 * OW + (OW - 1)], ev)

        pltpu.sync_copy(obuf.at[pl.ds(0, ORW)],
                        out_hbm.at[pl.ds(w * ORW, ORW)])

    return extract_kernel(dots_flat, exit_flat, tri)


def _ln(x, g, b):
    mu = jnp.mean(x, axis=-1, keepdims=True)
    xm = x - mu
    var = jnp.mean(xm * xm, axis=-1, keepdims=True)
    return xm * lax.rsqrt(var + 1e-5) * g + b


def _tc_dense(C2, emb_pad, gin_W1, gin_b1, gin_ln_g, gin_ln_b, gin_W2, gin_b2,
              post_W1, post_b1, post_W2, post_b2, norm_g, norm_b,
              exit_W1, exit_b1, exit_ln_g, exit_ln_b, exit_w2row, exit_b2v):
    GB = 16
    ROWS = GB * NN

    def body(C_ref, emb_ref, W1_ref, b1_ref, lg_ref, lb_ref, W2_ref, b2_ref,
             pW1_ref, pb1_ref, pW2_ref, pb2_ref, ng_ref, nb_ref,
             eW1_ref, eb1_ref, eg_ref, ebb_ref, ew2_ref, eb2_ref,
             dots_ref, exit_ref, A1_ref):
        @pl.when(pl.program_id(0) == 0)
        def _():
            A1_ref[...] = jnp.dot(emb_ref[...], W1_ref[...],
                                  preferred_element_type=jnp.float32)
        def dotf(a, b):
            return jnp.dot(a, b, preferred_element_type=jnp.float32)

        Cb = C_ref[...]
        h = dotf(Cb, A1_ref[...]) + b1_ref[...]
        h = jnp.maximum(_ln(h, lg_ref[...], lb_ref[...]), 0.0)
        h = dotf(h, W2_ref[...]) + b2_ref[...]
        h = jnp.maximum(dotf(h, pW1_ref[...]) + pb1_ref[...], 0.0)
        h = dotf(h, pW2_ref[...]) + pb2_ref[...]
        x = _ln(h, ng_ref[...], nb_ref[...])

        ridx = lax.broadcasted_iota(jnp.int32, (GB, ROWS), 1)
        gidx = lax.broadcasted_iota(jnp.int32, (GB, ROWS), 0)
        S = jnp.where(ridx // NN == gidx, jnp.float32(1.0 / NN), jnp.float32(0.0))
        means = jnp.dot(S, x, preferred_element_type=jnp.float32)

        e = jnp.dot(means, eW1_ref[...], preferred_element_type=jnp.float32) + eb1_ref[...]
        e = jnp.maximum(_ln(e, eg_ref[...], ebb_ref[...]), 0.0)
        ex = jnp.sum(e * ew2_ref[...], axis=-1, keepdims=True) + eb2_ref[0:1, 0:1]
        exit_ref[...] = ex

        for g in range(GB):
            xg = lax.slice(x, (g * NN, 0), ((g + 1) * NN, D))
            xgp = jnp.concatenate([xg, jnp.zeros((D - NN, D), jnp.float32)], 0)
            dg = lax.dot_general(xg, xgp, (((1,), (1,)), ((), ())),
                                 preferred_element_type=jnp.float32)
            dots_ref[pl.ds(g * NN, NN), :] = dg

    wspec = pl.BlockSpec((D, D), lambda i: (0, 0))
    vspec = pl.BlockSpec((1, D), lambda i: (0, 0))
    return pl.pallas_call(
        body,
        grid=(NG // GB,),
        in_specs=[
            pl.BlockSpec((ROWS, D), lambda i: (i, 0)),
            wspec, wspec, vspec, vspec, vspec, wspec, vspec,
            wspec, vspec, wspec, vspec, vspec, vspec,
            wspec, vspec, vspec, vspec, vspec, vspec,
        ],
        out_specs=[
            pl.BlockSpec((GB * NN, D), lambda i: (i, 0)),
            pl.BlockSpec((GB, 1), lambda i: (i, 0)),
        ],
        out_shape=[
            jax.ShapeDtypeStruct((NT, D), jnp.float32),
            jax.ShapeDtypeStruct((NG, 1), jnp.float32),
        ],
        scratch_shapes=[pltpu.VMEM((D, D), jnp.float32)],
    )(C2, emb_pad, gin_W1, gin_b1, gin_ln_g, gin_ln_b, gin_W2, gin_b2,
      post_W1, post_b1, post_W2, post_b2, norm_g, norm_b,
      exit_W1, exit_b1, exit_ln_g, exit_ln_b, exit_w2row, exit_b2v)


def kernel(node_ids, edge_index, ptr, emb, gin_W1, gin_b1, gin_ln_g, gin_ln_b,
           gin_W2, gin_b2, post_W1, post_b1, post_W2, post_b2, norm_g, norm_b,
           exit_W1, exit_b1, exit_ln_g, exit_ln_b, exit_W2, exit_b2):
    nids = node_ids.reshape(NT)

    keys = _sc_keys(nids, edge_index)
    C_flat = _sc_histogram(keys)
    C2 = C_flat.reshape(NT, D)

    emb_pad = jnp.zeros((D, D), jnp.float32).at[:NN].set(emb)
    r = lambda v: v.reshape(1, D)
    dots3, exit_out = _tc_dense(
        C2, emb_pad, gin_W1, r(gin_b1), r(gin_ln_g), r(gin_ln_b), gin_W2,
        r(gin_b2), post_W1, r(post_b1), post_W2, r(post_b2), r(norm_g),
        r(norm_b), exit_W1, r(exit_b1), r(exit_ln_g), r(exit_ln_b),
        exit_W2.reshape(1, D), jnp.full((1, D), exit_b2[0], jnp.float32))

    out_flat = _sc_extract(dots3.reshape(NT * D), exit_out.reshape(NG),
                           jnp.asarray(TRI_IDX))
    return out_flat.reshape(NG, OW)


# speedup vs baseline: 1.2359x; 1.0663x over previous
import functools

import numpy as np
import jax
import jax.numpy as jnp
from jax import lax
from jax.experimental import pallas as pl
from jax.experimental.pallas import tpu as pltpu
from jax.experimental.pallas import tpu_sc as plsc

NG = 512
NN = 100
NT = NG * NN
NE = NT * 16
D = 128

NC, NS = 2, 16
NW = NC * NS
HALF = NT // NC
CH = HALF // 2
CHW = CH * D
TRASH = CHW
SH_W = CHW + 2048
PTW = CHW // NS
ZB = 10240
EB_A = 6400
ESHARD = NE // NW
NSHARD = NT // NW
NK = NE + NT
KSHARD = NK // NS
KB = 3200
KR = KB // 128
NKB = KSHARD // KB

_IU, _JU = np.triu_indices(NN, k=1)
NTRI = _IU.size
OW = NTRI + 1
NTRI_PAD = 4960
TRI_IDX = np.zeros(NTRI_PAD, dtype=np.int32)
TRI_IDX[:NTRI] = _IU * D + _JU
GPT = NG // NW
ORW = GPT * OW


def _sc_keys(nids, edge_index):
    mesh = plsc.VectorSubcoreMesh(
        core_axis_name="c", subcore_axis_name="s", num_cores=NC, num_subcores=NS)

    @functools.partial(
        pl.kernel,
        out_type=jax.ShapeDtypeStruct((NK,), jnp.int32),
        mesh=mesh,
        compiler_params=pltpu.CompilerParams(needs_layout_passes=False),
        scratch_types=[
            pltpu.VMEM((NT,), jnp.int32),
            pltpu.VMEM((EB_A,), jnp.int32),
            pltpu.VMEM((EB_A,), jnp.int32),
            pltpu.VMEM((EB_A,), jnp.int32),
            pltpu.VMEM((EB_A,), jnp.int32),
            pltpu.VMEM((EB_A,), jnp.int32),
            pltpu.VMEM((EB_A,), jnp.int32),
            pltpu.SemaphoreType.DMA,
            pltpu.SemaphoreType.DMA,
        ],
    )
    def keys_kernel(nids_hbm, edge_hbm, out_hbm, nids_v, sbuf_a, sbuf_b,
                    dbuf_a, dbuf_b, kbuf_a, kbuf_b, sem_in, sem_out):
        c = lax.axis_index("c")
        s = lax.axis_index("s")
        w = s * NC + c
        iota16 = lax.iota(jnp.int32, 16)
        sbufs = (sbuf_a, sbuf_b)
        dbufs = (dbuf_a, dbuf_b)
        kbufs = (kbuf_a, kbuf_b)

        pltpu.sync_copy(nids_hbm, nids_v)

        NB_A = ESHARD // EB_A

        def eload(b):
            ebase = w * ESHARD + b * EB_A
            return (pltpu.async_copy(edge_hbm.at[0, pl.ds(ebase, EB_A)],
                                     sbufs[b % 2], sem_in),
                    pltpu.async_copy(edge_hbm.at[1, pl.ds(ebase, EB_A)],
                                     dbufs[b % 2], sem_in))

        lds = [None, None]
        sts = [None, None]
        lds[0] = eload(0)
        for b in range(NB_A):
            if b + 1 < NB_A:
                lds[(b + 1) % 2] = eload(b + 1)
            for dsc in lds[b % 2]:
                dsc.wait()
            if sts[b % 2] is not None:
                sts[b % 2].wait()
            sb, db, kb = sbufs[b % 2], dbufs[b % 2], kbufs[b % 2]

            @plsc.parallel_loop(0, EB_A // 16, unroll=8)
            def _(v):
                sv = sb[pl.ds(v * 16, 16)]
                dv = db[pl.ds(v * 16, 16)]
                nv = plsc.load_gather(nids_v, [sv])
                kb[pl.ds(v * 16, 16)] = dv * D + nv
            ebase = w * ESHARD + b * EB_A
            sts[b % 2] = pltpu.async_copy(
                kb, out_hbm.at[pl.ds(ebase, EB_A)], sem_out)

        nbase = w * NSHARD
        kb = kbufs[NB_A % 2]
        if sts[NB_A % 2] is not None:
            sts[NB_A % 2].wait()

        @plsc.parallel_loop(0, NSHARD // 16, unroll=8)
        def _(v):
            iv = nbase + v * 16 + iota16
            nv = plsc.load_gather(nids_v, [iv])
            kb[pl.ds(v * 16, 16)] = iv * D + nv
        pltpu.sync_copy(kb.at[pl.ds(0, NSHARD)],
                        out_hbm.at[pl.ds(NE + nbase, NSHARD)])
        for st in sts:
            if st is not None and st is not sts[NB_A % 2]:
                st.wait()

    return keys_kernel(nids, edge_index)


def _sc_histogram(keys):
    mesh = plsc.VectorSubcoreMesh(
        core_axis_name="c", subcore_axis_name="s", num_cores=NC, num_subcores=NS)

    @functools.partial(
        pl.kernel,
        out_type=jax.ShapeDtypeStruct((NT * D,), jnp.float32),
        mesh=mesh,
        compiler_params=pltpu.CompilerParams(needs_layout_passes=False),
        scratch_types=[
            pltpu.VMEM((KB,), jnp.int32),
            pltpu.VMEM((KB,), jnp.int32),
            pltpu.VMEM((KB,), jnp.int32),
            pltpu.VMEM((KB,), jnp.int32),
            pltpu.VMEM((KB,), jnp.float32),
            pltpu.VMEM((ZB,), jnp.float32),
            pltpu.VMEM_SHARED((SH_W,), jnp.float32),
            pltpu.SemaphoreType.DMA,
            pltpu.SemaphoreType.DMA,
            pltpu.SemaphoreType.DMA,
        ],
    )
    def hist_kernel(keys_hbm, out_hbm, kbuf_a, kbuf_b, idx_a, idx_b, ones1d,
                    zeros_v, shared, sem, sem_io, sem_k):
        c = lax.axis_index("c")
        s = lax.axis_index("s")

        def ones_body(v, _):
            ones1d[pl.ds(v * 16, 16)] = jnp.ones((16,), jnp.float32)
            return 0
        lax.fori_loop(0, KB // 16, ones_body, 0)

        def zbuf_body(i, _):
            zeros_v[pl.ds(i * 16, 16)] = jnp.zeros((16,), jnp.float32)
            return 0
        lax.fori_loop(0, ZB // 16, zbuf_body, 0)

        dump_d = None
        for chunk in range(2):
            lo = (c * HALF + chunk * CH) * D
            if dump_d is not None:
                dump_d.wait()
            zds = [pltpu.async_copy(zeros_v,
                                    shared.at[pl.ds(s * PTW + z * ZB, ZB)],
                                    sem_io)
                   for z in range(PTW // ZB)]
            for zd in zds:
                zd.wait()
            plsc.subcore_barrier()

            def kload(b):
                kbase = s * KSHARD + b * KB
                return pltpu.async_copy(keys_hbm.at[pl.ds(kbase, KB)],
                                        kbuf_a if b % 2 == 0 else kbuf_b,
                                        sem_k)

            descs = [None, None]
            lds = [None, None]
            lds[0] = kload(0)
            for b in range(NKB):
                if b + 1 < NKB:
                    lds[(b + 1) % 2] = kload(b + 1)
                lds[b % 2].wait()
                kb = kbuf_a if b % 2 == 0 else kbuf_b
                ib = idx_a if b % 2 == 0 else idx_b
                if descs[b % 2] is not None:
                    descs[b % 2].wait()

                @plsc.parallel_loop(0, KB // 16, unroll=8)
                def _(v):
                    kv = kb[pl.ds(v * 16, 16)]
                    rel = kv - lo
                    ok = (rel >= 0) & (rel < CHW)
                    ib[pl.ds(v * 16, 16)] = jnp.where(ok, rel, -1)
                descs[b % 2] = pltpu.async_copy(
                    ones1d,
                    shared.at[plsc.Indices(ib, ignored_value=-1)],
                    sem, add=True)
            for dsc in descs:
                if dsc is not None:
                    dsc.wait()

            plsc.subcore_barrier()
            dump_d = pltpu.async_copy(shared.at[pl.ds(s * PTW, PTW)],
                                      out_hbm.at[pl.ds(lo + s * PTW, PTW)],
                                      sem_io)
        dump_d.wait()

    return hist_kernel(keys)


def _sc_extract(dots_flat, exit_flat, tri):
    mesh = plsc.VectorSubcoreMesh(
        core_axis_name="c", subcore_axis_name="s", num_cores=NC, num_subcores=NS)

    @functools.partial(
        pl.kernel,
        out_type=jax.ShapeDtypeStruct((NG * OW,), jnp.float32),
        mesh=mesh,
        compiler_params=pltpu.CompilerParams(needs_layout_passes=False),
        scratch_types=[
            pltpu.VMEM((NN * D,), jnp.float32),
            pltpu.VMEM((NN * D,), jnp.float32),
            pltpu.VMEM((NTRI_PAD,), jnp.int32),
            pltpu.VMEM((NG,), jnp.float32),
            pltpu.VMEM((ORW + 16,), jnp.float32),
            pltpu.SemaphoreType.DMA,
            pltpu.SemaphoreType.DMA,
        ],
    )
    def extract_kernel(dots_hbm, exit_hbm, tri_hbm, out_hbm,
                       dbuf_a, dbuf_b, tri_v, exit_v, obuf, sem_a, sem_b):
        c = lax.axis_index("c")
        s = lax.axis_index("s")
        w = s * NC + c
        iota16 = lax.iota(jnp.int32, 16)
        dbufs = (dbuf_a, dbuf_b)
        sems = (sem_a, sem_b)

        pltpu.sync_copy(tri_hbm, tri_v)
        pltpu.sync_copy(exit_hbm, exit_v)

        def stage(g):
            gg = w * GPT + g
            return pltpu.async_copy(dots_hbm.at[pl.ds(gg * NN * D, NN * D)],
                                    dbufs[g % 2], sems[g % 2])

        stage_d = [None, None]
        stage_d[0] = stage(0)
        for g in range(GPT):
            if g + 1 < GPT:
                stage_d[(g + 1) % 2] = stage(g + 1)
            stage_d[g % 2].wait()
            dbuf = dbufs[g % 2]

            @plsc.parallel_loop(0, NTRI_PAD // 16, unroll=8)
            def _(k):
                tv = tri_v[pl.ds(k * 16, 16)]
                obuf[pl.ds(g * OW + k * 16, 16)] = plsc.load_gather(dbuf, [tv])

        ev = plsc.load_gather(exit_v, [w * GPT + iota16])
        plsc.store_scatter(obuf, [iota16 * OW + (OW - 1)], ev)

        pltpu.sync_copy(obuf.at[pl.ds(0, ORW)],
                        out_hbm.at[pl.ds(w * ORW, ORW)])

    return extract_kernel(dots_flat, exit_flat, tri)


def _ln(x, g, b):
    mu = jnp.mean(x, axis=-1, keepdims=True)
    xm = x - mu
    var = jnp.mean(xm * xm, axis=-1, keepdims=True)
    return xm * lax.rsqrt(var + 1e-5) * g + b


def _tc_dense(C2, emb_pad, gin_W1, gin_b1, gin_ln_g, gin_ln_b, gin_W2, gin_b2,
              post_W1, post_b1, post_W2, post_b2, norm_g, norm_b,
              exit_W1, exit_b1, exit_ln_g, exit_ln_b, exit_w2row, exit_b2v):
    GB = 32
    ROWS = GB * NN

    def body(C_ref, emb_ref, W1_ref, b1_ref, lg_ref, lb_ref, W2_ref, b2_ref,
             pW1_ref, pb1_ref, pW2_ref, pb2_ref, ng_ref, nb_ref,
             eW1_ref, eb1_ref, eg_ref, ebb_ref, ew2_ref, eb2_ref,
             dots_ref, exit_ref, A1_ref):
        @pl.when(pl.program_id(0) == 0)
        def _():
            A1_ref[...] = jnp.dot(emb_ref[...], W1_ref[...],
                                  preferred_element_type=jnp.float32)
        def dotf(a, b):
            return jnp.dot(a, b, preferred_element_type=jnp.float32)

        Cb = C_ref[...]
        h = dotf(Cb, A1_ref[...]) + b1_ref[...]
        h = jnp.maximum(_ln(h, lg_ref[...], lb_ref[...]), 0.0)
        h = dotf(h, W2_ref[...]) + b2_ref[...]
        h = jnp.maximum(dotf(h, pW1_ref[...]) + pb1_ref[...], 0.0)
        h = dotf(h, pW2_ref[...]) + pb2_ref[...]
        x = _ln(h, ng_ref[...], nb_ref[...])

        ridx = lax.broadcasted_iota(jnp.int32, (GB, ROWS), 1)
        gidx = lax.broadcasted_iota(jnp.int32, (GB, ROWS), 0)
        S = jnp.where(ridx // NN == gidx, jnp.float32(1.0 / NN), jnp.float32(0.0))
        means = jnp.dot(S, x, preferred_element_type=jnp.float32)

        e = jnp.dot(means, eW1_ref[...], preferred_element_type=jnp.float32) + eb1_ref[...]
        e = jnp.maximum(_ln(e, eg_ref[...], ebb_ref[...]), 0.0)
        ex = jnp.sum(e * ew2_ref[...], axis=-1, keepdims=True) + eb2_ref[0:1, 0:1]
        exit_ref[...] = ex

        for g in range(GB):
            xg = lax.slice(x, (g * NN, 0), ((g + 1) * NN, D))
            xgp = jnp.concatenate([xg, jnp.zeros((D - NN, D), jnp.float32)], 0)
            dg = lax.dot_general(xg, xgp, (((1,), (1,)), ((), ())),
                                 preferred_element_type=jnp.float32)
            dots_ref[pl.ds(g * NN, NN), :] = dg

    wspec = pl.BlockSpec((D, D), lambda i: (0, 0))
    vspec = pl.BlockSpec((1, D), lambda i: (0, 0))
    return pl.pallas_call(
        body,
        grid=(NG // GB,),
        in_specs=[
            pl.BlockSpec((ROWS, D), lambda i: (i, 0)),
            wspec, wspec, vspec, vspec, vspec, wspec, vspec,
            wspec, vspec, wspec, vspec, vspec, vspec,
            wspec, vspec, vspec, vspec, vspec, vspec,
        ],
        out_specs=[
            pl.BlockSpec((GB * NN, D), lambda i: (i, 0)),
            pl.BlockSpec((GB, 1), lambda i: (i, 0)),
        ],
        out_shape=[
            jax.ShapeDtypeStruct((NT, D), jnp.float32),
            jax.ShapeDtypeStruct((NG, 1), jnp.float32),
        ],
        scratch_shapes=[pltpu.VMEM((D, D), jnp.float32)],
    )(C2, emb_pad, gin_W1, gin_b1, gin_ln_g, gin_ln_b, gin_W2, gin_b2,
      post_W1, post_b1, post_W2, post_b2, norm_g, norm_b,
      exit_W1, exit_b1, exit_ln_g, exit_ln_b, exit_w2row, exit_b2v)


def kernel(node_ids, edge_index, ptr, emb, gin_W1, gin_b1, gin_ln_g, gin_ln_b,
           gin_W2, gin_b2, post_W1, post_b1, post_W2, post_b2, norm_g, norm_b,
           exit_W1, exit_b1, exit_ln_g, exit_ln_b, exit_W2, exit_b2):
    nids = node_ids.reshape(NT)

    keys = _sc_keys(nids, edge_index)
    C_flat = _sc_histogram(keys)
    C2 = C_flat.reshape(NT, D)

    emb_pad = jnp.zeros((D, D), jnp.float32).at[:NN].set(emb)
    r = lambda v: v.reshape(1, D)
    dots3, exit_out = _tc_dense(
        C2, emb_pad, gin_W1, r(gin_b1), r(gin_ln_g), r(gin_ln_b), gin_W2,
        r(gin_b2), post_W1, r(post_b1), post_W2, r(post_b2), r(norm_g),
        r(norm_b), exit_W1, r(exit_b1), r(exit_ln_g), r(exit_ln_b),
        exit_W2.reshape(1, D), jnp.full((1, D), exit_b2[0], jnp.float32))

    out_flat = _sc_extract(dots3.reshape(NT * D), exit_out.reshape(NG),
                           jnp.asarray(TRI_IDX))
    return out_flat.reshape(NG, OW)


# speedup vs baseline: 1.2585x; 1.0183x over previous
import functools

import numpy as np
import jax
import jax.numpy as jnp
from jax import lax
from jax.experimental import pallas as pl
from jax.experimental.pallas import tpu as pltpu
from jax.experimental.pallas import tpu_sc as plsc

NG = 512
NN = 100
NT = NG * NN
NE = NT * 16
D = 128

NC, NS = 2, 16
NW = NC * NS
HALF = NT // NC
CH = HALF // 2
CHW = CH * D
TRASH = CHW
SH_W = CHW + 2048
PTW = CHW // NS
ZB = 10240
EB_A = 6400
ESHARD = NE // NW
NSHARD = NT // NW
NK = NE + NT
KSHARD = NK // NS
KB = 3200
KR = KB // 128
NKB = KSHARD // KB

_IU, _JU = np.triu_indices(NN, k=1)
NTRI = _IU.size
OW = NTRI + 1
NTRI_PAD = 4960
TRI_IDX = np.zeros(NTRI_PAD, dtype=np.int32)
TRI_IDX[:NTRI] = _IU * D + _JU
GPT = NG // NW
ORW = GPT * OW


def _sc_keys(nids, edge_index):
    mesh = plsc.VectorSubcoreMesh(
        core_axis_name="c", subcore_axis_name="s", num_cores=NC, num_subcores=NS)

    @functools.partial(
        pl.kernel,
        out_type=jax.ShapeDtypeStruct((NK,), jnp.int32),
        mesh=mesh,
        compiler_params=pltpu.CompilerParams(needs_layout_passes=False),
        scratch_types=[
            pltpu.VMEM((NT,), jnp.int32),
            pltpu.VMEM((EB_A,), jnp.int32),
            pltpu.VMEM((EB_A,), jnp.int32),
            pltpu.VMEM((EB_A,), jnp.int32),
            pltpu.VMEM((EB_A,), jnp.int32),
            pltpu.VMEM((EB_A,), jnp.int32),
            pltpu.VMEM((EB_A,), jnp.int32),
            pltpu.SemaphoreType.DMA,
            pltpu.SemaphoreType.DMA,
        ],
    )
    def keys_kernel(nids_hbm, edge_hbm, out_hbm, nids_v, sbuf_a, sbuf_b,
                    dbuf_a, dbuf_b, kbuf_a, kbuf_b, sem_in, sem_out):
        c = lax.axis_index("c")
        s = lax.axis_index("s")
        w = s * NC + c
        iota16 = lax.iota(jnp.int32, 16)
        sbufs = (sbuf_a, sbuf_b)
        dbufs = (dbuf_a, dbuf_b)
        kbufs = (kbuf_a, kbuf_b)

        pltpu.sync_copy(nids_hbm, nids_v)

        NB_A = ESHARD // EB_A

        def eload(b):
            ebase = w * ESHARD + b * EB_A
            return (pltpu.async_copy(edge_hbm.at[0, pl.ds(ebase, EB_A)],
                                     sbufs[b % 2], sem_in),
                    pltpu.async_copy(edge_hbm.at[1, pl.ds(ebase, EB_A)],
                                     dbufs[b % 2], sem_in))

        lds = [None, None]
        sts = [None, None]
        lds[0] = eload(0)
        for b in range(NB_A):
            if b + 1 < NB_A:
                lds[(b + 1) % 2] = eload(b + 1)
            for dsc in lds[b % 2]:
                dsc.wait()
            if sts[b % 2] is not None:
                sts[b % 2].wait()
            sb, db, kb = sbufs[b % 2], dbufs[b % 2], kbufs[b % 2]

            @plsc.parallel_loop(0, EB_A // 16, unroll=8)
            def _(v):
                sv = sb[pl.ds(v * 16, 16)]
                dv = db[pl.ds(v * 16, 16)]
                nv = plsc.load_gather(nids_v, [sv])
                kb[pl.ds(v * 16, 16)] = dv * D + nv
            ebase = w * ESHARD + b * EB_A
            sts[b % 2] = pltpu.async_copy(
                kb, out_hbm.at[pl.ds(ebase, EB_A)], sem_out)

        nbase = w * NSHARD
        kb = kbufs[NB_A % 2]
        if sts[NB_A % 2] is not None:
            sts[NB_A % 2].wait()

        @plsc.parallel_loop(0, NSHARD // 16, unroll=8)
        def _(v):
            iv = nbase + v * 16 + iota16
            nv = plsc.load_gather(nids_v, [iv])
            kb[pl.ds(v * 16, 16)] = iv * D + nv
        pltpu.sync_copy(kb.at[pl.ds(0, NSHARD)],
                        out_hbm.at[pl.ds(NE + nbase, NSHARD)])
        for st in sts:
            if st is not None and st is not sts[NB_A % 2]:
                st.wait()

    return keys_kernel(nids, edge_index)


def _sc_histogram(keys):
    mesh = plsc.VectorSubcoreMesh(
        core_axis_name="c", subcore_axis_name="s", num_cores=NC, num_subcores=NS)

    @functools.partial(
        pl.kernel,
        out_type=jax.ShapeDtypeStruct((NT * D,), jnp.float32),
        mesh=mesh,
        compiler_params=pltpu.CompilerParams(needs_layout_passes=False),
        scratch_types=[
            pltpu.VMEM((KB,), jnp.int32),
            pltpu.VMEM((KB,), jnp.int32),
            pltpu.VMEM((KB,), jnp.int32),
            pltpu.VMEM((KB,), jnp.int32),
            pltpu.VMEM((KB,), jnp.float32),
            pltpu.VMEM((ZB,), jnp.float32),
            pltpu.VMEM_SHARED((SH_W,), jnp.float32),
            pltpu.SemaphoreType.DMA,
            pltpu.SemaphoreType.DMA,
            pltpu.SemaphoreType.DMA,
        ],
    )
    def hist_kernel(keys_hbm, out_hbm, kbuf_a, kbuf_b, idx_a, idx_b, ones1d,
                    zeros_v, shared, sem, sem_io, sem_k):
        c = lax.axis_index("c")
        s = lax.axis_index("s")

        def ones_body(v, _):
            ones1d[pl.ds(v * 16, 16)] = jnp.ones((16,), jnp.float32)
            return 0
        lax.fori_loop(0, KB // 16, ones_body, 0)

        def zbuf_body(i, _):
            zeros_v[pl.ds(i * 16, 16)] = jnp.zeros((16,), jnp.float32)
            return 0
        lax.fori_loop(0, ZB // 16, zbuf_body, 0)

        dump_d = None
        for chunk in range(2):
            lo = (c * HALF + chunk * CH) * D
            if dump_d is not None:
                dump_d.wait()
            zds = [pltpu.async_copy(zeros_v,
                                    shared.at[pl.ds(s * PTW + z * ZB, ZB)],
                                    sem_io)
                   for z in range(PTW // ZB)]
            for zd in zds:
                zd.wait()
            plsc.subcore_barrier()

            def kload(b):
                kbase = s * KSHARD + b * KB
                return pltpu.async_copy(keys_hbm.at[pl.ds(kbase, KB)],
                                        kbuf_a if b % 2 == 0 else kbuf_b,
                                        sem_k)

            descs = [None, None]
            lds = [None, None]
            lds[0] = kload(0)
            for b in range(NKB):
                if b + 1 < NKB:
                    lds[(b + 1) % 2] = kload(b + 1)
                lds[b % 2].wait()
                kb = kbuf_a if b % 2 == 0 else kbuf_b
                ib = idx_a if b % 2 == 0 else idx_b
                if descs[b % 2] is not None:
                    descs[b % 2].wait()

                @plsc.parallel_loop(0, KB // 16, unroll=8)
                def _(v):
                    kv = kb[pl.ds(v * 16, 16)]
                    rel = kv - lo
                    ok = (rel >= 0) & (rel < CHW)
                    ib[pl.ds(v * 16, 16)] = jnp.where(ok, rel, -1)
                descs[b % 2] = pltpu.async_copy(
                    ones1d,
                    shared.at[plsc.Indices(ib, ignored_value=-1)],
                    sem, add=True)
            for dsc in descs:
                if dsc is not None:
                    dsc.wait()

            plsc.subcore_barrier()
            dump_d = pltpu.async_copy(shared.at[pl.ds(s * PTW, PTW)],
                                      out_hbm.at[pl.ds(lo + s * PTW, PTW)],
                                      sem_io)
        dump_d.wait()

    return hist_kernel(keys)


def _sc_extract(dots_flat, exit_flat, tri):
    mesh = plsc.VectorSubcoreMesh(
        core_axis_name="c", subcore_axis_name="s", num_cores=NC, num_subcores=NS)

    @functools.partial(
        pl.kernel,
        out_type=jax.ShapeDtypeStruct((NG * OW,), jnp.float32),
        mesh=mesh,
        compiler_params=pltpu.CompilerParams(needs_layout_passes=False),
        scratch_types=[
            pltpu.VMEM((NN * D,), jnp.float32),
            pltpu.VMEM((NN * D,), jnp.float32),
            pltpu.VMEM((NTRI_PAD,), jnp.int32),
            pltpu.VMEM((NG,), jnp.float32),
            pltpu.VMEM((ORW + 16,), jnp.float32),
            pltpu.SemaphoreType.DMA,
            pltpu.SemaphoreType.DMA,
        ],
    )
    def extract_kernel(dots_hbm, exit_hbm, tri_hbm, out_hbm,
                       dbuf_a, dbuf_b, tri_v, exit_v, obuf, sem_a, sem_b):
        c = lax.axis_index("c")
        s = lax.axis_index("s")
        w = s * NC + c
        iota16 = lax.iota(jnp.int32, 16)
        dbufs = (dbuf_a, dbuf_b)
        sems = (sem_a, sem_b)

        pltpu.sync_copy(tri_hbm, tri_v)
        pltpu.sync_copy(exit_hbm, exit_v)

        def stage(g):
            gg = w * GPT + g
            return pltpu.async_copy(dots_hbm.at[pl.ds(gg * NN * D, NN * D)],
                                    dbufs[g % 2], sems[g % 2])

        stage_d = [None, None]
        stage_d[0] = stage(0)
        for g in range(GPT):
            if g + 1 < GPT:
                stage_d[(g + 1) % 2] = stage(g + 1)
            stage_d[g % 2].wait()
            dbuf = dbufs[g % 2]

            @plsc.parallel_loop(0, NTRI_PAD // 16, unroll=8)
            def _(k):
                tv = tri_v[pl.ds(k * 16, 16)]
                obuf[pl.ds(g * OW + k * 16, 16)] = plsc.load_gather(dbuf, [tv])

        ev = plsc.load_gather(exit_v, [w * GPT + iota16])
        plsc.store_scatter(obuf, [iota16 * OW + (OW - 1)], ev)

        pltpu.sync_copy(obuf.at[pl.ds(0, ORW)],
                        out_hbm.at[pl.ds(w * ORW, ORW)])

    return extract_kernel(dots_flat, exit_flat, tri)


def _ln(x, g, b):
    mu = jnp.mean(x, axis=-1, keepdims=True)
    xm = x - mu
    var = jnp.mean(xm * xm, axis=-1, keepdims=True)
    return xm * lax.rsqrt(var + 1e-5) * g + b


def _tc_dense(C2, emb_pad, gin_W1, gin_b1, gin_ln_g, gin_ln_b, gin_W2, gin_b2,
              post_W1, post_b1, post_W2, post_b2, norm_g, norm_b,
              exit_W1, exit_b1, exit_ln_g, exit_ln_b, exit_w2row, exit_b2v):
    GB = 64
    ROWS = GB * NN

    def body(C_ref, emb_ref, W1_ref, b1_ref, lg_ref, lb_ref, W2_ref, b2_ref,
             pW1_ref, pb1_ref, pW2_ref, pb2_ref, ng_ref, nb_ref,
             eW1_ref, eb1_ref, eg_ref, ebb_ref, ew2_ref, eb2_ref,
             dots_ref, exit_ref, A1_ref):
        @pl.when(pl.program_id(0) == 0)
        def _():
            A1_ref[...] = jnp.dot(emb_ref[...], W1_ref[...],
                                  preferred_element_type=jnp.float32)
        def dotf(a, b):
            return jnp.dot(a, b, preferred_element_type=jnp.float32)

        Cb = C_ref[...]
        h = dotf(Cb, A1_ref[...]) + b1_ref[...]
        h = jnp.maximum(_ln(h, lg_ref[...], lb_ref[...]), 0.0)
        h = dotf(h, W2_ref[...]) + b2_ref[...]
        h = jnp.maximum(dotf(h, pW1_ref[...]) + pb1_ref[...], 0.0)
        h = dotf(h, pW2_ref[...]) + pb2_ref[...]
        x = _ln(h, ng_ref[...], nb_ref[...])

        ridx = lax.broadcasted_iota(jnp.int32, (GB, ROWS), 1)
        gidx = lax.broadcasted_iota(jnp.int32, (GB, ROWS), 0)
        S = jnp.where(ridx // NN == gidx, jnp.float32(1.0 / NN), jnp.float32(0.0))
        means = jnp.dot(S, x, preferred_element_type=jnp.float32)

        e = jnp.dot(means, eW1_ref[...], preferred_element_type=jnp.float32) + eb1_ref[...]
        e = jnp.maximum(_ln(e, eg_ref[...], ebb_ref[...]), 0.0)
        ex = jnp.sum(e * ew2_ref[...], axis=-1, keepdims=True) + eb2_ref[0:1, 0:1]
        exit_ref[...] = ex

        for g in range(GB):
            xg = lax.slice(x, (g * NN, 0), ((g + 1) * NN, D))
            xgp = jnp.concatenate([xg, jnp.zeros((D - NN, D), jnp.float32)], 0)
            dg = lax.dot_general(xg, xgp, (((1,), (1,)), ((), ())),
                                 preferred_element_type=jnp.float32)
            dots_ref[pl.ds(g * NN, NN), :] = dg

    wspec = pl.BlockSpec((D, D), lambda i: (0, 0))
    vspec = pl.BlockSpec((1, D), lambda i: (0, 0))
    return pl.pallas_call(
        body,
        grid=(NG // GB,),
        in_specs=[
            pl.BlockSpec((ROWS, D), lambda i: (i, 0)),
            wspec, wspec, vspec, vspec, vspec, wspec, vspec,
            wspec, vspec, wspec, vspec, vspec, vspec,
            wspec, vspec, vspec, vspec, vspec, vspec,
        ],
        out_specs=[
            pl.BlockSpec((GB * NN, D), lambda i: (i, 0)),
            pl.BlockSpec((GB, 1), lambda i: (i, 0)),
        ],
        out_shape=[
            jax.ShapeDtypeStruct((NT, D), jnp.float32),
            jax.ShapeDtypeStruct((NG, 1), jnp.float32),
        ],
        scratch_shapes=[pltpu.VMEM((D, D), jnp.float32)],
    )(C2, emb_pad, gin_W1, gin_b1, gin_ln_g, gin_ln_b, gin_W2, gin_b2,
      post_W1, post_b1, post_W2, post_b2, norm_g, norm_b,
      exit_W1, exit_b1, exit_ln_g, exit_ln_b, exit_w2row, exit_b2v)


def kernel(node_ids, edge_index, ptr, emb, gin_W1, gin_b1, gin_ln_g, gin_ln_b,
           gin_W2, gin_b2, post_W1, post_b1, post_W2, post_b2, norm_g, norm_b,
           exit_W1, exit_b1, exit_ln_g, exit_ln_b, exit_W2, exit_b2):
    nids = node_ids.reshape(NT)

    keys = _sc_keys(nids, edge_index)
    C_flat = _sc_histogram(keys)
    C2 = C_flat.reshape(NT, D)

    emb_pad = jnp.zeros((D, D), jnp.float32).at[:NN].set(emb)
    r = lambda v: v.reshape(1, D)
    dots3, exit_out = _tc_dense(
        C2, emb_pad, gin_W1, r(gin_b1), r(gin_ln_g), r(gin_ln_b), gin_W2,
        r(gin_b2), post_W1, r(post_b1), post_W2, r(post_b2), r(norm_g),
        r(norm_b), exit_W1, r(exit_b1), r(exit_ln_g), r(exit_ln_b),
        exit_W2.reshape(1, D), jnp.full((1, D), exit_b2[0], jnp.float32))

    out_flat = _sc_extract(dots3.reshape(NT * D), exit_out.reshape(NG),
                           jnp.asarray(TRI_IDX))
    return out_flat.reshape(NG, OW)
